# baseline (speedup 1.0000x reference)
import functools
import numpy as np
import jax
import jax.numpy as jnp
from jax import lax
from jax.experimental import pallas as pl
from jax.experimental.pallas import tpu as pltpu
from jax.experimental.pallas import tpu_sc as plsc

NC = 2
NS = 16
NW = NC * NS
CH = 128
GR = 1
NG = 80
ZR = 32

_CBN = float(1.0 / np.sqrt(1.0 + 1e-5))


def _sc_mesh():
    return plsc.VectorSubcoreMesh(
        core_axis_name="c", subcore_axis_name="s", num_cores=NC, num_subcores=NS
    )


def _make_deg_kernel(np_pad):
    rpw = np_pad // NS

    @functools.partial(
        pl.kernel,
        out_type=jax.ShapeDtypeStruct((NC * np_pad, 128), jnp.float32),
        mesh=_sc_mesh(),
        scratch_types=[
            pltpu.VMEM_SHARED((np_pad, 128), jnp.float32),
            pltpu.VMEM((2 * GR, CH), jnp.int32),
            pltpu.VMEM((CH, 128), jnp.float32),
            pltpu.VMEM((ZR, 128), jnp.float32),
        ],
    )
    def deg_kernel(esd_hbm, out_hbm, acc_sh, ib, ones_v, zv):
        c = lax.axis_index("c")
        s = lax.axis_index("s")
        for r in range(CH):
            for k in range(128 // 16):
                ones_v[r, pl.ds(k * 16, 16)] = jnp.ones((16,), jnp.float32)
        for r in range(ZR):
            for k in range(128 // 16):
                zv[r, pl.ds(k * 16, 16)] = jnp.zeros((16,), jnp.float32)
        rowbase = s * rpw
        for i in range(rpw // ZR):
            pltpu.sync_copy(zv, acc_sh.at[pl.ds(rowbase + i * ZR, ZR)])
        plsc.subcore_barrier()
        gbase = (c * NS + s) * NG

        def group(t, carry):
            pltpu.sync_copy(esd_hbm.at[gbase + t], ib)
            for q in range(GR):
                pltpu.sync_copy(ones_v, acc_sh.at[ib.at[2 * q + 1]], add=True)
            return carry

        lax.fori_loop(0, NG, group, 0)
        plsc.subcore_barrier()
        pltpu.sync_copy(
            acc_sh.at[pl.ds(rowbase, rpw)],
            out_hbm.at[pl.ds(c * np_pad + rowbase, rpw)],
        )

    return deg_kernel


def _make_agg_kernel(np_pad, h):
    rpw = np_pad // NS

    @functools.partial(
        pl.kernel,
        out_type=jax.ShapeDtypeStruct((NC * np_pad, h), jnp.float32),
        mesh=_sc_mesh(),
        scratch_types=[
            pltpu.VMEM_SHARED((np_pad, h), jnp.float32),
            pltpu.VMEM((2 * GR, CH), jnp.int32),
            pltpu.VMEM((CH, h), jnp.float32),
            pltpu.VMEM((ZR, h), jnp.float32),
            pltpu.SemaphoreType.DMA,
        ],
    )
    def agg_kernel(hp_hbm, esd_hbm, out_hbm, acc_sh, ib, rb0, zv, sg0):
        rbufs = (rb0,)
        sgs = (sg0,)
        c = lax.axis_index("c")
        s = lax.axis_index("s")
        for r in range(ZR):
            for k in range(h // 16):
                zv[r, pl.ds(k * 16, 16)] = jnp.zeros((16,), jnp.float32)
        rowbase = s * rpw
        for i in range(rpw // ZR):
            pltpu.sync_copy(zv, acc_sh.at[pl.ds(rowbase + i * ZR, ZR)])
        plsc.subcore_barrier()
        gbase = (c * NS + s) * NG

        def group(t, carry):
            pltpu.sync_copy(esd_hbm.at[gbase + t], ib)
            gathers = [
                pltpu.async_copy(hp_hbm.at[ib.at[2 * q]], rbufs[q], sgs[q])
                for q in range(GR)
            ]
            for q in range(GR):
                gathers[q].wait()
                pltpu.sync_copy(rbufs[q], acc_sh.at[ib.at[2 * q + 1]], add=True)
            return carry

        lax.fori_loop(0, NG, group, 0)
        plsc.subcore_barrier()
        pltpu.sync_copy(
            acc_sh.at[pl.ds(rowbase, rpw)],
            out_hbm.at[pl.ds(c * np_pad + rowbase, rpw)],
        )

    return agg_kernel


def _mm1_body(x_ref, w_ref, degp_ref, h_ref, dis_ref):
    deg = degp_ref[0, :, :1] + degp_ref[1, :, :1] + 1.0
    dis = lax.rsqrt(deg)
    hm = jnp.dot(x_ref[...], w_ref[...], preferred_element_type=jnp.float32)
    h_ref[...] = hm * dis
    dis_ref[...] = dis


def _mid_body(acc_ref, hp_ref, dis_ref, b_ref, g_ref, be_ref, w_ref, out_ref):
    dis = dis_ref[...]
    pre = (acc_ref[0] + acc_ref[1] + hp_ref[...]) * dis + b_ref[...]
    y = jnp.maximum(pre * (g_ref[...] * _CBN) + be_ref[...], 0.0)
    out_ref[...] = (
        jnp.dot(y, w_ref[...], preferred_element_type=jnp.float32) * dis
    )


def _pool_body(acc_ref, hp_ref, dis_ref, b_ref, g_ref, be_ref, bt_ref,
               sum_ref, max_ref, cnt_ref, *, block_rows):
    @pl.when(pl.program_id(0) == 0)
    def _init():
        sum_ref[...] = jnp.zeros_like(sum_ref)
        max_ref[...] = jnp.full_like(max_ref, -jnp.inf)
        cnt_ref[...] = jnp.zeros_like(cnt_ref)

    pre = (acc_ref[0] + acc_ref[1] + hp_ref[...]) * dis_ref[...] + b_ref[...]
    y = jnp.maximum(pre * (g_ref[...] * _CBN) + be_ref[...], 0.0)
    bt = bt_ref[...]
    lo = bt_ref[0, 0]
    hi = bt_ref[block_rows - 1, 0]

    def seg(sid, carry):
        m = bt == sid
        ssum = jnp.sum(jnp.where(m, y, 0.0), axis=0, keepdims=True)
        smax = jnp.max(jnp.where(m, y, -jnp.inf), axis=0, keepdims=True)
        scnt = jnp.sum(m.astype(jnp.float32), axis=0, keepdims=True)
        sum_ref[pl.ds(sid, 1), :] += ssum
        max_ref[pl.ds(sid, 1), :] = jnp.maximum(max_ref[pl.ds(sid, 1), :], smax)
        cnt_ref[pl.ds(sid, 1), :] += scnt
        return carry

    lax.fori_loop(lo, hi + 1, seg, 0)


def _mlp_body(sum_ref, max_ref, cnt_ref, w1_ref, b1_ref, w2_ref, b2_ref,
              w3_ref, b3_ref, out_ref):
    mean = sum_ref[...] / jnp.maximum(cnt_ref[...], 1.0)
    hcat = jnp.concatenate([mean, max_ref[...]], axis=1)
    h1 = jnp.maximum(
        jnp.dot(hcat, w1_ref[...], preferred_element_type=jnp.float32) + b1_ref[...], 0.0
    )
    h2 = jnp.maximum(
        jnp.dot(h1, w2_ref[...], preferred_element_type=jnp.float32) + b2_ref[...], 0.0
    )
    out_ref[...] = (
        jnp.dot(h2, w3_ref[...], preferred_element_type=jnp.float32) + b3_ref[...]
    )


def kernel(x, edge_index, batch, W1, b1, W2, b2, W3, b3, g1, be1, g2, be2, g3, be3, mW1, mb1, mW2, mb2, mW3, mb3):
    n, d = x.shape
    h = W1.shape[1]
    G = 64
    e = edge_index.shape[1]

    ep = NW * NG * GR * CH
    np_unit = NS * ZR
    np_pad = ((n + np_unit - 1) // np_unit) * np_unit
    pad = ep - e
    src = jnp.concatenate([edge_index[0], jnp.zeros((pad,), jnp.int32)])
    dst = jnp.concatenate([edge_index[1], jnp.full((pad,), n, jnp.int32)])
    esd = jnp.stack(
        [src.reshape(NW, NG, GR, CH), dst.reshape(NW, NG, GR, CH)], axis=3
    ).reshape(NW * NG, 2 * GR, CH)

    deg_k = _make_deg_kernel(np_pad)
    agg_k = _make_agg_kernel(np_pad, h)

    degp = deg_k(esd).reshape(NC, np_pad, 128)

    B = 1000
    grid = n // B
    f32 = jnp.float32

    row_spec = pl.BlockSpec((B, h), lambda i: (i, 0))
    dis_spec = pl.BlockSpec((B, 1), lambda i: (i, 0))
    acc_spec = pl.BlockSpec((NC, B, h), lambda i: (0, i, 0))
    vec_spec = pl.BlockSpec((1, h), lambda i: (0, 0))
    w_spec = pl.BlockSpec((h, h), lambda i: (0, 0))

    h1p, dis = pl.pallas_call(
        _mm1_body,
        grid=(grid,),
        in_specs=[
            pl.BlockSpec((B, d), lambda i: (i, 0)),
            pl.BlockSpec((d, h), lambda i: (0, 0)),
            pl.BlockSpec((NC, B, 128), lambda i: (0, i, 0)),
        ],
        out_specs=[row_spec, dis_spec],
        out_shape=[
            jax.ShapeDtypeStruct((n, h), f32),
            jax.ShapeDtypeStruct((n, 1), f32),
        ],
    )(x, W1, degp)

    def mid(accp, hp, bl, gl, bel, Wn):
        return pl.pallas_call(
            _mid_body,
            grid=(grid,),
            in_specs=[acc_spec, row_spec, dis_spec, vec_spec, vec_spec, vec_spec, w_spec],
            out_specs=row_spec,
            out_shape=jax.ShapeDtypeStruct((n, h), f32),
        )(accp, hp, dis, bl.reshape(1, h), gl.reshape(1, h), bel.reshape(1, h), Wn)

    accp1 = agg_k(h1p, esd).reshape(NC, np_pad, h)
    h2p = mid(accp1, h1p, b1, g1, be1, W2)
    accp2 = agg_k(h2p, esd).reshape(NC, np_pad, h)
    h3p = mid(accp2, h2p, b2, g2, be2, W3)
    accp3 = agg_k(h3p, esd).reshape(NC, np_pad, h)

    sums, maxs, cnt = pl.pallas_call(
        functools.partial(_pool_body, block_rows=B),
        grid=(grid,),
        in_specs=[
            acc_spec, row_spec, dis_spec, vec_spec, vec_spec, vec_spec,
            pl.BlockSpec((B, 1), lambda i: (i, 0)),
        ],
        out_specs=[
            pl.BlockSpec((G, h), lambda i: (0, 0)),
            pl.BlockSpec((G, h), lambda i: (0, 0)),
            pl.BlockSpec((G, 1), lambda i: (0, 0)),
        ],
        out_shape=[
            jax.ShapeDtypeStruct((G, h), f32),
            jax.ShapeDtypeStruct((G, h), f32),
            jax.ShapeDtypeStruct((G, 1), f32),
        ],
    )(accp3, h3p, dis, b3.reshape(1, h), g3.reshape(1, h), be3.reshape(1, h),
      batch.reshape(n, 1))

    out = pl.pallas_call(
        _mlp_body,
        out_shape=jax.ShapeDtypeStruct((G, 1), f32),
    )(sums, maxs, cnt, mW1, mb1.reshape(1, -1), mW2, mb2.reshape(1, -1),
      mW3, mb3.reshape(1, 1))

    return out.reshape(G)


# speedup vs baseline: 1.2460x; 1.2460x over previous
import functools
import numpy as np
import jax
import jax.numpy as jnp
from jax import lax
from jax.experimental import pallas as pl
from jax.experimental.pallas import tpu as pltpu
from jax.experimental.pallas import tpu_sc as plsc

NC = 2
NS = 16
NW = NC * NS
CH = 128
NCHUNK = 79
ZR = 64

_CBN = float(1.0 / np.sqrt(1.0 + 1e-5))


def _sc_mesh():
    return plsc.VectorSubcoreMesh(
        core_axis_name="c", subcore_axis_name="s", num_cores=NC, num_subcores=NS
    )


def _make_deg_kernel(np_pad):
    rpw = np_pad // NS
    epw = NCHUNK * CH

    @functools.partial(
        pl.kernel,
        out_type=jax.ShapeDtypeStruct((NC * np_pad, 128), jnp.float32),
        mesh=_sc_mesh(),
        scratch_types=[
            pltpu.VMEM_SHARED((np_pad, 128), jnp.float32),
            pltpu.VMEM((CH,), jnp.int32),
            pltpu.VMEM((CH, 128), jnp.float32),
            pltpu.VMEM((ZR, 128), jnp.float32),
        ],
    )
    def deg_kernel(dst_hbm, out_hbm, acc_sh, idx_d, ones_v, zv):
        c = lax.axis_index("c")
        s = lax.axis_index("s")
        for r in range(CH):
            for k in range(128 // 16):
                ones_v[r, pl.ds(k * 16, 16)] = jnp.ones((16,), jnp.float32)
        for r in range(ZR):
            for k in range(128 // 16):
                zv[r, pl.ds(k * 16, 16)] = jnp.zeros((16,), jnp.float32)
        rowbase = s * rpw
        for i in range(rpw // ZR):
            pltpu.sync_copy(zv, acc_sh.at[pl.ds(rowbase + i * ZR, ZR)])
        plsc.subcore_barrier()
        ebase = (c * NS + s) * epw

        def chunk(i, carry):
            off = ebase + i * CH
            pltpu.sync_copy(dst_hbm.at[pl.ds(off, CH)], idx_d)
            pltpu.sync_copy(ones_v, acc_sh.at[idx_d], add=True)
            return carry

        lax.fori_loop(0, NCHUNK, chunk, 0)
        plsc.subcore_barrier()
        pltpu.sync_copy(
            acc_sh.at[pl.ds(rowbase, rpw)],
            out_hbm.at[pl.ds(c * np_pad + rowbase, rpw)],
        )

    return deg_kernel


def _make_agg_kernel(np_pad, h):
    rpw = np_pad // NS
    epw = NCHUNK * CH

    @functools.partial(
        pl.kernel,
        out_type=jax.ShapeDtypeStruct((NC * np_pad, h), jnp.float32),
        mesh=_sc_mesh(),
        scratch_types=[
            pltpu.VMEM_SHARED((np_pad, h), jnp.float32),
            pltpu.VMEM((CH,), jnp.int32),
            pltpu.VMEM((CH,), jnp.int32),
            pltpu.VMEM((CH, h), jnp.float32),
            pltpu.VMEM((ZR, h), jnp.float32),
            pltpu.SemaphoreType.DMA,
        ],
    )
    def agg_kernel(hp_hbm, src_hbm, dst_hbm, out_hbm, acc_sh, idx_s, idx_d,
                   rows, zv, sem):
        c = lax.axis_index("c")
        s = lax.axis_index("s")
        for r in range(ZR):
            for k in range(h // 16):
                zv[r, pl.ds(k * 16, 16)] = jnp.zeros((16,), jnp.float32)
        rowbase = s * rpw
        for i in range(rpw // ZR):
            pltpu.sync_copy(zv, acc_sh.at[pl.ds(rowbase + i * ZR, ZR)])
        plsc.subcore_barrier()
        ebase = (c * NS + s) * epw

        def chunk(i, carry):
            off = ebase + i * CH
            pltpu.sync_copy(src_hbm.at[pl.ds(off, CH)], idx_s)
            pltpu.sync_copy(dst_hbm.at[pl.ds(off, CH)], idx_d)
            pltpu.async_copy(hp_hbm.at[idx_s], rows, sem).wait()
            pltpu.sync_copy(rows, acc_sh.at[idx_d], add=True)
            return carry

        lax.fori_loop(0, NCHUNK, chunk, 0)
        plsc.subcore_barrier()
        pltpu.sync_copy(
            acc_sh.at[pl.ds(rowbase, rpw)],
            out_hbm.at[pl.ds(c * np_pad + rowbase, rpw)],
        )

    return agg_kernel


def _mm1a_body(x_ref, w_ref, out_ref):
    out_ref[...] = jnp.dot(x_ref[...], w_ref[...], preferred_element_type=jnp.float32)


def _scale_body(degp_ref, hraw_ref, h_ref, dis_ref):
    deg = degp_ref[0, :, :1] + degp_ref[1, :, :1] + 1.0
    dis = lax.rsqrt(deg)
    h_ref[...] = hraw_ref[...] * dis
    dis_ref[...] = dis


def _mid_body(acc_ref, hp_ref, dis_ref, b_ref, g_ref, be_ref, w_ref, out_ref):
    dis = dis_ref[...]
    pre = (acc_ref[0] + acc_ref[1] + hp_ref[...]) * dis + b_ref[...]
    y = jnp.maximum(pre * (g_ref[...] * _CBN) + be_ref[...], 0.0)
    out_ref[...] = (
        jnp.dot(y, w_ref[...], preferred_element_type=jnp.float32) * dis
    )


def _pool_body(acc_ref, hp_ref, dis_ref, b_ref, g_ref, be_ref, bt_ref,
               sum_ref, max_ref, cnt_ref, *, block_rows):
    @pl.when(pl.program_id(0) == 0)
    def _init():
        sum_ref[...] = jnp.zeros_like(sum_ref)
        max_ref[...] = jnp.full_like(max_ref, -jnp.inf)
        cnt_ref[...] = jnp.zeros_like(cnt_ref)

    pre = (acc_ref[0] + acc_ref[1] + hp_ref[...]) * dis_ref[...] + b_ref[...]
    y = jnp.maximum(pre * (g_ref[...] * _CBN) + be_ref[...], 0.0)
    bt = bt_ref[...]
    lo = bt_ref[0, 0]
    hi = bt_ref[block_rows - 1, 0]

    def seg(sid, carry):
        m = bt == sid
        ssum = jnp.sum(jnp.where(m, y, 0.0), axis=0, keepdims=True)
        smax = jnp.max(jnp.where(m, y, -jnp.inf), axis=0, keepdims=True)
        scnt = jnp.sum(m.astype(jnp.float32), axis=0, keepdims=True)
        sum_ref[pl.ds(sid, 1), :] += ssum
        max_ref[pl.ds(sid, 1), :] = jnp.maximum(max_ref[pl.ds(sid, 1), :], smax)
        cnt_ref[pl.ds(sid, 1), :] += scnt
        return carry

    lax.fori_loop(lo, hi + 1, seg, 0)


def _mlp_body(sum_ref, max_ref, cnt_ref, w1_ref, b1_ref, w2_ref, b2_ref,
              w3_ref, b3_ref, out_ref):
    mean = sum_ref[...] / jnp.maximum(cnt_ref[...], 1.0)
    hcat = jnp.concatenate([mean, max_ref[...]], axis=1)
    h1 = jnp.maximum(
        jnp.dot(hcat, w1_ref[...], preferred_element_type=jnp.float32) + b1_ref[...], 0.0
    )
    h2 = jnp.maximum(
        jnp.dot(h1, w2_ref[...], preferred_element_type=jnp.float32) + b2_ref[...], 0.0
    )
    out_ref[...] = (
        jnp.dot(h2, w3_ref[...], preferred_element_type=jnp.float32) + b3_ref[...]
    )


def kernel(x, edge_index, batch, W1, b1, W2, b2, W3, b3, g1, be1, g2, be2, g3, be3, mW1, mb1, mW2, mb2, mW3, mb3):
    n, d = x.shape
    h = W1.shape[1]
    G = 64
    e = edge_index.shape[1]

    ep = NW * NCHUNK * CH
    np_unit = NS * ZR
    np_pad = ((n + np_unit - 1) // np_unit) * np_unit
    pad = ep - e
    src = jnp.concatenate([edge_index[0], jnp.zeros((pad,), jnp.int32)])
    dst = jnp.concatenate([edge_index[1], jnp.full((pad,), n, jnp.int32)])

    deg_k = _make_deg_kernel(np_pad)
    agg_k = _make_agg_kernel(np_pad, h)

    degp = deg_k(dst).reshape(NC, np_pad, 128)

    B = 1000
    grid = n // B
    f32 = jnp.float32

    row_spec = pl.BlockSpec((B, h), lambda i: (i, 0))
    dis_spec = pl.BlockSpec((B, 1), lambda i: (i, 0))
    acc_spec = pl.BlockSpec((NC, B, h), lambda i: (0, i, 0))
    vec_spec = pl.BlockSpec((1, h), lambda i: (0, 0))
    w_spec = pl.BlockSpec((h, h), lambda i: (0, 0))

    h1raw = pl.pallas_call(
        _mm1a_body,
        grid=(grid,),
        in_specs=[
            pl.BlockSpec((B, d), lambda i: (i, 0)),
            pl.BlockSpec((d, h), lambda i: (0, 0)),
        ],
        out_specs=row_spec,
        out_shape=jax.ShapeDtypeStruct((n, h), f32),
    )(x, W1)

    h1p, dis = pl.pallas_call(
        _scale_body,
        grid=(grid,),
        in_specs=[
            pl.BlockSpec((NC, B, 128), lambda i: (0, i, 0)),
            row_spec,
        ],
        out_specs=[row_spec, dis_spec],
        out_shape=[
            jax.ShapeDtypeStruct((n, h), f32),
            jax.ShapeDtypeStruct((n, 1), f32),
        ],
    )(degp, h1raw)

    def mid(accp, hp, bl, gl, bel, Wn):
        return pl.pallas_call(
            _mid_body,
            grid=(grid,),
            in_specs=[acc_spec, row_spec, dis_spec, vec_spec, vec_spec, vec_spec, w_spec],
            out_specs=row_spec,
            out_shape=jax.ShapeDtypeStruct((n, h), f32),
        )(accp, hp, dis, bl.reshape(1, h), gl.reshape(1, h), bel.reshape(1, h), Wn)

    accp1 = agg_k(h1p, src, dst).reshape(NC, np_pad, h)
    h2p = mid(accp1, h1p, b1, g1, be1, W2)
    accp2 = agg_k(h2p, src, dst).reshape(NC, np_pad, h)
    h3p = mid(accp2, h2p, b2, g2, be2, W3)
    accp3 = agg_k(h3p, src, dst).reshape(NC, np_pad, h)

    sums, maxs, cnt = pl.pallas_call(
        functools.partial(_pool_body, block_rows=B),
        grid=(grid,),
        in_specs=[
            acc_spec, row_spec, dis_spec, vec_spec, vec_spec, vec_spec,
            pl.BlockSpec((B, 1), lambda i: (i, 0)),
        ],
        out_specs=[
            pl.BlockSpec((G, h), lambda i: (0, 0)),
            pl.BlockSpec((G, h), lambda i: (0, 0)),
            pl.BlockSpec((G, 1), lambda i: (0, 0)),
        ],
        out_shape=[
            jax.ShapeDtypeStruct((G, h), f32),
            jax.ShapeDtypeStruct((G, h), f32),
            jax.ShapeDtypeStruct((G, 1), f32),
        ],
    )(accp3, h3p, dis, b3.reshape(1, h), g3.reshape(1, h), be3.reshape(1, h),
      batch.reshape(n, 1))

    out = pl.pallas_call(
        _mlp_body,
        out_shape=jax.ShapeDtypeStruct((G, 1), f32),
    )(sums, maxs, cnt, mW1, mb1.reshape(1, -1), mW2, mb2.reshape(1, -1),
      mW3, mb3.reshape(1, 1))

    return out.reshape(G)


# speedup vs baseline: 1.3620x; 1.0931x over previous
import functools
import numpy as np
import jax
import jax.numpy as jnp
from jax import lax
from jax.experimental import pallas as pl
from jax.experimental.pallas import tpu as pltpu
from jax.experimental.pallas import tpu_sc as plsc

NC = 2
NS = 16
NW = NC * NS
CH = 128
NCHUNK = 79
ZR = 64

_CBN = float(1.0 / np.sqrt(1.0 + 1e-5))


def _sc_mesh():
    return plsc.VectorSubcoreMesh(
        core_axis_name="c", subcore_axis_name="s", num_cores=NC, num_subcores=NS
    )


def _make_deg_kernel(np_pad):
    rpw = np_pad // NS
    epw = NCHUNK * CH

    @functools.partial(
        pl.kernel,
        out_type=jax.ShapeDtypeStruct((NC * np_pad, 128), jnp.float32),
        mesh=_sc_mesh(),
        scratch_types=[
            pltpu.VMEM_SHARED((np_pad, 128), jnp.float32),
            pltpu.VMEM((CH,), jnp.int32),
            pltpu.VMEM((CH, 128), jnp.float32),
            pltpu.VMEM((ZR, 128), jnp.float32),
        ],
    )
    def deg_kernel(dst_hbm, out_hbm, acc_sh, idx_d, ones_v, zv):
        c = lax.axis_index("c")
        s = lax.axis_index("s")
        for r in range(CH):
            for k in range(128 // 16):
                ones_v[r, pl.ds(k * 16, 16)] = jnp.ones((16,), jnp.float32)
        for r in range(ZR):
            for k in range(128 // 16):
                zv[r, pl.ds(k * 16, 16)] = jnp.zeros((16,), jnp.float32)
        rowbase = s * rpw
        for i in range(rpw // ZR):
            pltpu.sync_copy(zv, acc_sh.at[pl.ds(rowbase + i * ZR, ZR)])
        plsc.subcore_barrier()
        ebase = (c * NS + s) * epw

        def chunk(i, carry):
            off = ebase + i * CH
            pltpu.sync_copy(dst_hbm.at[pl.ds(off, CH)], idx_d)
            pltpu.sync_copy(ones_v, acc_sh.at[idx_d], add=True)
            return carry

        lax.fori_loop(0, NCHUNK, chunk, 0)
        plsc.subcore_barrier()
        pltpu.sync_copy(
            acc_sh.at[pl.ds(rowbase, rpw)],
            out_hbm.at[pl.ds(c * np_pad + rowbase, rpw)],
        )

    return deg_kernel


def _make_agg_kernel(np_pad, h):
    rpw = np_pad // NS
    epw = NCHUNK * CH

    @functools.partial(
        pl.kernel,
        out_type=jax.ShapeDtypeStruct((NC * np_pad, h), jnp.float32),
        mesh=_sc_mesh(),
        scratch_types=[
            pltpu.VMEM_SHARED((np_pad, h), jnp.float32),
            pltpu.VMEM((CH,), jnp.int32),
            pltpu.VMEM((CH,), jnp.int32),
            pltpu.VMEM((CH, h), jnp.float32),
            pltpu.VMEM((ZR, h), jnp.float32),
            pltpu.SemaphoreType.DMA,
        ],
    )
    def agg_kernel(hp_hbm, src_hbm, dst_hbm, out_hbm, acc_sh, idx_s, idx_d,
                   rows, zv, sem):
        c = lax.axis_index("c")
        s = lax.axis_index("s")
        for r in range(ZR):
            for k in range(h // 16):
                zv[r, pl.ds(k * 16, 16)] = jnp.zeros((16,), jnp.float32)
        rowbase = s * rpw
        for i in range(rpw // ZR):
            pltpu.sync_copy(zv, acc_sh.at[pl.ds(rowbase + i * ZR, ZR)])
        plsc.subcore_barrier()
        ebase = (c * NS + s) * epw

        def chunk(i, carry):
            off = ebase + i * CH
            pltpu.sync_copy(src_hbm.at[pl.ds(off, CH)], idx_s)
            pltpu.sync_copy(dst_hbm.at[pl.ds(off, CH)], idx_d)
            pltpu.async_copy(hp_hbm.at[idx_s], rows, sem).wait()
            pltpu.sync_copy(rows, acc_sh.at[idx_d], add=True)
            return carry

        lax.fori_loop(0, NCHUNK, chunk, 0)
        plsc.subcore_barrier()
        pltpu.sync_copy(
            acc_sh.at[pl.ds(rowbase, rpw)],
            out_hbm.at[pl.ds(c * np_pad + rowbase, rpw)],
        )

    return agg_kernel


def _mm1_body(x_ref, w_ref, degp_ref, h_ref, dis_ref):
    deg = degp_ref[0, :, :1] + degp_ref[1, :, :1] + 1.0
    dis = lax.rsqrt(deg)
    hm = jnp.dot(x_ref[...], w_ref[...], preferred_element_type=jnp.float32)
    h_ref[...] = hm * dis
    dis_ref[...] = dis


def _mid_body(acc_ref, hp_ref, dis_ref, b_ref, g_ref, be_ref, w_ref, out_ref):
    dis = dis_ref[...]
    pre = (acc_ref[0] + acc_ref[1] + hp_ref[...]) * dis + b_ref[...]
    y = jnp.maximum(pre * (g_ref[...] * _CBN) + be_ref[...], 0.0)
    out_ref[...] = (
        jnp.dot(y, w_ref[...], preferred_element_type=jnp.float32) * dis
    )


def _pool_body(acc_ref, hp_ref, dis_ref, b_ref, g_ref, be_ref, bt_ref,
               sum_ref, max_ref, cnt_ref, *, block_rows):
    @pl.when(pl.program_id(0) == 0)
    def _init():
        sum_ref[...] = jnp.zeros_like(sum_ref)
        max_ref[...] = jnp.full_like(max_ref, -jnp.inf)
        cnt_ref[...] = jnp.zeros_like(cnt_ref)

    pre = (acc_ref[0] + acc_ref[1] + hp_ref[...]) * dis_ref[...] + b_ref[...]
    y = jnp.maximum(pre * (g_ref[...] * _CBN) + be_ref[...], 0.0)
    bt = bt_ref[...]
    lo = bt_ref[0, 0]
    hi = bt_ref[block_rows - 1, 0]

    def seg(sid, carry):
        m = bt == sid
        ssum = jnp.sum(jnp.where(m, y, 0.0), axis=0, keepdims=True)
        smax = jnp.max(jnp.where(m, y, -jnp.inf), axis=0, keepdims=True)
        scnt = jnp.sum(m.astype(jnp.float32), axis=0, keepdims=True)
        sum_ref[pl.ds(sid, 1), :] += ssum
        max_ref[pl.ds(sid, 1), :] = jnp.maximum(max_ref[pl.ds(sid, 1), :], smax)
        cnt_ref[pl.ds(sid, 1), :] += scnt
        return carry

    lax.fori_loop(lo, hi + 1, seg, 0)


def _mlp_body(sum_ref, max_ref, cnt_ref, w1_ref, b1_ref, w2_ref, b2_ref,
              w3_ref, b3_ref, out_ref):
    mean = sum_ref[...] / jnp.maximum(cnt_ref[...], 1.0)
    hcat = jnp.concatenate([mean, max_ref[...]], axis=1)
    h1 = jnp.maximum(
        jnp.dot(hcat, w1_ref[...], preferred_element_type=jnp.float32) + b1_ref[...], 0.0
    )
    h2 = jnp.maximum(
        jnp.dot(h1, w2_ref[...], preferred_element_type=jnp.float32) + b2_ref[...], 0.0
    )
    out_ref[...] = (
        jnp.dot(h2, w3_ref[...], preferred_element_type=jnp.float32) + b3_ref[...]
    )


def kernel(x, edge_index, batch, W1, b1, W2, b2, W3, b3, g1, be1, g2, be2, g3, be3, mW1, mb1, mW2, mb2, mW3, mb3):
    n, d = x.shape
    h = W1.shape[1]
    G = 64
    e = edge_index.shape[1]

    ep = NW * NCHUNK * CH
    np_unit = NS * ZR
    np_pad = ((n + np_unit - 1) // np_unit) * np_unit
    pad = ep - e
    src = jnp.concatenate([edge_index[0], jnp.zeros((pad,), jnp.int32)])
    dst = jnp.concatenate([edge_index[1], jnp.full((pad,), n, jnp.int32)])

    deg_k = _make_deg_kernel(np_pad)
    agg_k = _make_agg_kernel(np_pad, h)

    degp = deg_k(dst).reshape(NC, np_pad, 128)

    B = 1000
    grid = n // B
    f32 = jnp.float32

    row_spec = pl.BlockSpec((B, h), lambda i: (i, 0))
    dis_spec = pl.BlockSpec((B, 1), lambda i: (i, 0))
    acc_spec = pl.BlockSpec((NC, B, h), lambda i: (0, i, 0))
    vec_spec = pl.BlockSpec((1, h), lambda i: (0, 0))
    w_spec = pl.BlockSpec((h, h), lambda i: (0, 0))

    h1p, dis = pl.pallas_call(
        _mm1_body,
        grid=(grid,),
        in_specs=[
            pl.BlockSpec((B, d), lambda i: (i, 0)),
            pl.BlockSpec((d, h), lambda i: (0, 0)),
            pl.BlockSpec((NC, B, 128), lambda i: (0, i, 0)),
        ],
        out_specs=[row_spec, dis_spec],
        out_shape=[
            jax.ShapeDtypeStruct((n, h), f32),
            jax.ShapeDtypeStruct((n, 1), f32),
        ],
    )(x, W1, degp)

    def mid(accp, hp, bl, gl, bel, Wn):
        return pl.pallas_call(
            _mid_body,
            grid=(grid,),
            in_specs=[acc_spec, row_spec, dis_spec, vec_spec, vec_spec, vec_spec, w_spec],
            out_specs=row_spec,
            out_shape=jax.ShapeDtypeStruct((n, h), f32),
        )(accp, hp, dis, bl.reshape(1, h), gl.reshape(1, h), bel.reshape(1, h), Wn)

    accp1 = agg_k(h1p, src, dst).reshape(NC, np_pad, h)
    h2p = mid(accp1, h1p, b1, g1, be1, W2)
    accp2 = agg_k(h2p, src, dst).reshape(NC, np_pad, h)
    h3p = mid(accp2, h2p, b2, g2, be2, W3)
    accp3 = agg_k(h3p, src, dst).reshape(NC, np_pad, h)

    sums, maxs, cnt = pl.pallas_call(
        functools.partial(_pool_body, block_rows=B),
        grid=(grid,),
        in_specs=[
            acc_spec, row_spec, dis_spec, vec_spec, vec_spec, vec_spec,
            pl.BlockSpec((B, 1), lambda i: (i, 0)),
        ],
        out_specs=[
            pl.BlockSpec((G, h), lambda i: (0, 0)),
            pl.BlockSpec((G, h), lambda i: (0, 0)),
            pl.BlockSpec((G, 1), lambda i: (0, 0)),
        ],
        out_shape=[
            jax.ShapeDtypeStruct((G, h), f32),
            jax.ShapeDtypeStruct((G, h), f32),
            jax.ShapeDtypeStruct((G, 1), f32),
        ],
    )(accp3, h3p, dis, b3.reshape(1, h), g3.reshape(1, h), be3.reshape(1, h),
      batch.reshape(n, 1))

    out = pl.pallas_call(
        _mlp_body,
        out_shape=jax.ShapeDtypeStruct((G, 1), f32),
    )(sums, maxs, cnt, mW1, mb1.reshape(1, -1), mW2, mb2.reshape(1, -1),
      mW3, mb3.reshape(1, 1))

    return out.reshape(G)
